# baseline (speedup 1.0000x reference)
import functools

import jax
import jax.numpy as jnp
from jax import lax
from jax.experimental import pallas as pl
from jax.experimental.pallas import tpu as pltpu
from jax.experimental.pallas import tpu_sc as plsc

B, C, V = 4096, 200, 128
NT, NE = 128, 64
NC, NS, LANES = 2, 16, 16
NW = NC * NS
ROWS = B * C
RPW = ROWS // NW
CHUNK = 128
NCHUNK = RPW // CHUNK
NBUF = 4
GROUPS = NCHUNK // NBUF

CB = 8


NCOPY = 2


def _build_body(tm_ref, mm_ref, tab_ref, cm_ref, out_ref):
    i = pl.program_id(1)
    tm = tm_ref[...]
    mm = mm_ref[...]
    onehot = (tm[:, None] == lax.broadcasted_iota(jnp.int32, (NT, NE), 1))
    fused = lax.dot(onehot.astype(jnp.float32), tab_ref[...],
                    precision=lax.Precision.HIGHEST,
                    preferred_element_type=jnp.float32)
    fused = fused * mm[:, None]
    cpos = (lax.broadcasted_iota(jnp.int32, (NT, CB, V), 1) + i * CB).astype(jnp.float32)
    masked = cm_ref[...][None, None, :] != 0
    out_ref[...] = jnp.where(masked, cpos, fused[:, None, :])[None]


def _build_expanded(token_map, mult_map, table, channel_mask):
    return pl.pallas_call(
        _build_body,
        grid=(NCOPY, C // CB),
        in_specs=[
            pl.BlockSpec((NT,), lambda k, i: (0,)),
            pl.BlockSpec((NT,), lambda k, i: (0,)),
            pl.BlockSpec((NE, V), lambda k, i: (0, 0)),
            pl.BlockSpec((V,), lambda k, i: (0,)),
        ],
        out_specs=pl.BlockSpec((1, NT, CB, V), lambda k, i: (k, 0, i, 0)),
        out_shape=jax.ShapeDtypeStruct((NCOPY, NT, C, V), jnp.float32),
    )(token_map, mult_map, table, channel_mask.astype(jnp.int32))


def _sc_body(exp_hbm, inp_hbm, out_hbm, idx2,
             b0, b1, b2, b3, g0, g1, g2, g3, s0, s1, s2, s3):
    bufs = (b0, b1, b2, b3)
    gsems = (g0, g1, g2, g3)
    ssems = (s0, s1, s2, s3)
    wid = lax.axis_index("s") * NC + lax.axis_index("c")

    pltpu.sync_copy(inp_hbm.at[wid], idx2)

    iota16 = lax.broadcasted_iota(jnp.int32, (LANES,), 0)
    par = (iota16 & 1) * (NT * C)

    def prep(g):
        for j in range(CHUNK // LANES):
            raw = idx2[g, pl.ds(j * LANES, LANES)]
            pos = lax.rem(g * CHUNK + j * LANES + iota16, C)
            idx2[g, pl.ds(j * LANES, LANES)] = raw * C + pos + par

    def fire_gather(g, b):
        prep(g)
        pltpu.async_copy(exp_hbm.at[idx2.at[g]], bufs[b], gsems[b])

    def wait_gather(g, b):
        pltpu.make_async_copy(exp_hbm.at[idx2.at[g]], bufs[b], gsems[b]).wait()

    def fire_scatter(g, b):
        pltpu.async_copy(bufs[b], out_hbm.at[wid, g], ssems[b])

    def wait_scatter(g, b):
        pltpu.make_async_copy(bufs[b], out_hbm.at[wid, g], ssems[b]).wait()

    for b in range(NBUF):
        fire_gather(b, b)

    def group(p, carry):
        for b in range(NBUF):
            g = p * NBUF + b
            wait_gather(g, b)
            fire_scatter(g, b)
        for b in range(NBUF):
            g = p * NBUF + b
            wait_scatter(g, b)
            fire_gather(g + NBUF, b)
        return carry

    lax.fori_loop(0, GROUPS - 1, group, 0)

    last = (GROUPS - 1) * NBUF
    for b in range(NBUF):
        wait_gather(last + b, b)
        fire_scatter(last + b, b)
    for b in range(NBUF):
        wait_scatter(last + b, b)


@functools.partial(jax.jit, static_argnames=())
def kernel(input_BC, token_map, mult_map, table, channel_mask):
    expanded = _build_expanded(token_map, mult_map, table, channel_mask)
    exp2d = expanded.reshape(NCOPY * NT * C, V)
    inp3d = input_BC.reshape(NW, NCHUNK, CHUNK)

    gather = pl.kernel(
        _sc_body,
        out_type=jax.ShapeDtypeStruct((NW, NCHUNK, CHUNK, V), jnp.float32),
        mesh=plsc.VectorSubcoreMesh(core_axis_name="c", subcore_axis_name="s"),
        scratch_types=[
            pltpu.VMEM((NCHUNK, CHUNK), jnp.int32),
            pltpu.VMEM((CHUNK, V), jnp.float32),
            pltpu.VMEM((CHUNK, V), jnp.float32),
            pltpu.VMEM((CHUNK, V), jnp.float32),
            pltpu.VMEM((CHUNK, V), jnp.float32),
            pltpu.SemaphoreType.DMA,
            pltpu.SemaphoreType.DMA,
            pltpu.SemaphoreType.DMA,
            pltpu.SemaphoreType.DMA,
            pltpu.SemaphoreType.DMA,
            pltpu.SemaphoreType.DMA,
            pltpu.SemaphoreType.DMA,
            pltpu.SemaphoreType.DMA,
        ],
    )
    out4 = gather(exp2d, inp3d)
    return out4.reshape(B, C, V)


# speedup vs baseline: 1.0471x; 1.0471x over previous
import functools

import jax
import jax.numpy as jnp
from jax import lax
from jax.experimental import pallas as pl
from jax.experimental.pallas import tpu as pltpu
from jax.experimental.pallas import tpu_sc as plsc

B, C, V = 4096, 200, 128
NT, NE = 128, 64
NC, NS, LANES = 2, 16, 16
NW = NC * NS
ROWS = B * C
RPW = ROWS // NW
CHUNK = 128
NCHUNK = RPW // CHUNK
NBUF = 4
GROUPS = NCHUNK // NBUF

CB = 8


def _build_body(tm_ref, mm_ref, tab_ref, cm_ref, out_ref):
    i = pl.program_id(0)
    tm = tm_ref[...]
    mm = mm_ref[...]
    onehot = (tm[:, None] == lax.broadcasted_iota(jnp.int32, (NT, NE), 1))
    fused = lax.dot(onehot.astype(jnp.float32), tab_ref[...],
                    precision=lax.Precision.HIGHEST,
                    preferred_element_type=jnp.float32)
    fused = fused * mm[:, None]
    cpos = (lax.broadcasted_iota(jnp.int32, (NT, CB, V), 1) + i * CB).astype(jnp.float32)
    masked = cm_ref[...][None, None, :] != 0
    out_ref[...] = jnp.where(masked, cpos, fused[:, None, :])


def _build_expanded(token_map, mult_map, table, channel_mask):
    return pl.pallas_call(
        _build_body,
        grid=(C // CB,),
        in_specs=[
            pl.BlockSpec((NT,), lambda i: (0,)),
            pl.BlockSpec((NT,), lambda i: (0,)),
            pl.BlockSpec((NE, V), lambda i: (0, 0)),
            pl.BlockSpec((V,), lambda i: (0,)),
        ],
        out_specs=pl.BlockSpec((NT, CB, V), lambda i: (0, i, 0)),
        out_shape=jax.ShapeDtypeStruct((NT, C, V), jnp.float32),
    )(token_map, mult_map, table, channel_mask.astype(jnp.int32))


def _sc_body(exp_hbm, inp_hbm, out_hbm, idx2,
             b0, b1, b2, b3, g0, g1, g2, g3, s0, s1, s2, s3):
    bufs = (b0, b1, b2, b3)
    gsems = (g0, g1, g2, g3)
    ssems = (s0, s1, s2, s3)
    wid = lax.axis_index("s") * NC + lax.axis_index("c")

    pltpu.sync_copy(inp_hbm.at[wid], idx2)

    iota16 = lax.broadcasted_iota(jnp.int32, (LANES,), 0)

    def prep(g):
        for j in range(CHUNK // LANES):
            raw = idx2[g, pl.ds(j * LANES, LANES)]
            pos = lax.rem(g * CHUNK + j * LANES + iota16, C)
            idx2[g, pl.ds(j * LANES, LANES)] = raw * C + pos

    def fire_gather(g, b):
        prep(g)
        pltpu.async_copy(exp_hbm.at[idx2.at[g]], bufs[b], gsems[b])

    def wait_gather(g, b):
        pltpu.make_async_copy(exp_hbm.at[idx2.at[g]], bufs[b], gsems[b]).wait()

    def fire_scatter(g, b):
        pltpu.async_copy(bufs[b], out_hbm.at[wid, g], ssems[b])

    def wait_scatter(g, b):
        pltpu.make_async_copy(bufs[b], out_hbm.at[wid, g], ssems[b]).wait()

    for b in range(NBUF):
        fire_gather(b, b)

    def group(p, carry):
        for b in range(NBUF):
            g = p * NBUF + b
            wait_gather(g, b)
            fire_scatter(g, b)
        for b in range(NBUF):
            g = p * NBUF + b
            wait_scatter(g, b)
            fire_gather(g + NBUF, b)
        return carry

    lax.fori_loop(0, GROUPS - 1, group, 0)

    last = (GROUPS - 1) * NBUF
    for b in range(NBUF):
        wait_gather(last + b, b)
        fire_scatter(last + b, b)
    for b in range(NBUF):
        wait_scatter(last + b, b)


@functools.partial(jax.jit, static_argnames=())
def kernel(input_BC, token_map, mult_map, table, channel_mask):
    expanded = _build_expanded(token_map, mult_map, table, channel_mask)
    exp2d = expanded.reshape(NT * C, V)
    inp3d = input_BC.reshape(NW, NCHUNK, CHUNK)

    gather = pl.kernel(
        _sc_body,
        out_type=jax.ShapeDtypeStruct((NW, NCHUNK, CHUNK, V), jnp.float32),
        mesh=plsc.VectorSubcoreMesh(core_axis_name="c", subcore_axis_name="s"),
        scratch_types=[
            pltpu.VMEM((NCHUNK, CHUNK), jnp.int32),
            pltpu.VMEM((CHUNK, V), jnp.float32),
            pltpu.VMEM((CHUNK, V), jnp.float32),
            pltpu.VMEM((CHUNK, V), jnp.float32),
            pltpu.VMEM((CHUNK, V), jnp.float32),
            pltpu.SemaphoreType.DMA,
            pltpu.SemaphoreType.DMA,
            pltpu.SemaphoreType.DMA,
            pltpu.SemaphoreType.DMA,
            pltpu.SemaphoreType.DMA,
            pltpu.SemaphoreType.DMA,
            pltpu.SemaphoreType.DMA,
            pltpu.SemaphoreType.DMA,
        ],
    )
    out4 = gather(exp2d, inp3d)
    return out4.reshape(B, C, V)
